# Initial kernel scaffold; baseline (speedup 1.0000x reference)
import jax
import jax.numpy as jnp
from jax.experimental import pallas as pl


def kernel(item_seq, items_to_predict, table):
    raise NotImplementedError("write your pallas kernel here")


# speedup vs baseline: 1.2937x; 1.2937x over previous
import functools

import jax
import jax.numpy as jnp
from jax import lax
from jax.experimental import pallas as pl
from jax.experimental.pallas import tpu as pltpu
from jax.experimental.pallas import tpu_sc as plsc

B = 4096
HIST = 50
NPRED = 100
NPRED_PAD = 112
D = 64
NC = 2
NS = 16
NW = NC * NS
RPW = B // NW
NBUF = 4
NCH = D // 16


def _iota16():
    return lax.iota(jnp.int32, 16)


def _sc_body(table, seq, pidx, dot_out, nb2_out, na2_out,
             seq_v, pidx_v, hsum_v, tbuf, ubuf, dot_v, nb_v, na2_v,
             h0b, h1b, h2b, h3b, p0b, p1b, p2b, p3b,
             sem0, sem1, sem2, sem3):
    hembs = (h0b, h1b, h2b, h3b)
    pembs = (p0b, p1b, p2b, p3b)
    sems = (sem0, sem1, sem2, sem3)

    wid = lax.axis_index("s") * NC + lax.axis_index("c")
    base = wid * RPW

    pltpu.sync_copy(seq.at[pl.ds(base, RPW)], seq_v)
    pltpu.sync_copy(pidx.at[pl.ds(base, RPW)], pidx_v)

    zeros = jnp.zeros((16,), jnp.float32)
    for pb in pembs:
        for p in range(NPRED, NPRED_PAD):
            for c in range(NCH):
                pb[p, pl.ds(c * 16, 16)] = zeros

    def fire(r, b):
        pltpu.async_copy(table.at[seq_v.at[r]], hembs[b], sems[b])
        pltpu.async_copy(table.at[pidx_v.at[r]],
                         pembs[b].at[pl.ds(0, NPRED)], sems[b])

    for b in range(NBUF):
        fire(b, b)

    @pl.loop(0, RPW // NBUF)
    def _row_group(rg):
        for b in range(NBUF):
            r = rg * NBUF + b
            hemb = hembs[b]
            pemb = pembs[b]
            pltpu.make_async_copy(table.at[seq_v.at[r]], hemb,
                                  sems[b]).wait()
            pltpu.make_async_copy(table.at[pidx_v.at[r]],
                                  pemb.at[pl.ds(0, NPRED)], sems[b]).wait()

            def hist_step(i, acc):
                return tuple(acc[c] + hemb[i, pl.ds(c * 16, 16)]
                             for c in range(NCH))

            h = lax.fori_loop(0, HIST, hist_step, (zeros,) * NCH,
                              unroll=2)
            for c in range(NCH):
                hsum_v[r, pl.ds(c * 16, 16)] = h[c]

            @pl.loop(0, NPRED_PAD // 16)
            def _pred_group(pg):
                pbase = pg * 16
                for j in range(16):
                    p = pbase + j
                    e = [pemb[p, pl.ds(c * 16, 16)] for c in range(NCH)]
                    t = e[0] * h[0]
                    u = e[0] * e[0]
                    for c in range(1, NCH):
                        t = t + e[c] * h[c]
                        u = u + e[c] * e[c]
                    tbuf[j, pl.ds(0, 16)] = t
                    ubuf[j, pl.ds(0, 16)] = u
                dacc = zeros
                nacc = zeros
                rows = _iota16()
                for l in range(16):
                    col = jnp.full((16,), l, jnp.int32)
                    dacc = dacc + plsc.load_gather(tbuf, [rows, col])
                    nacc = nacc + plsc.load_gather(ubuf, [rows, col])
                dot_v[r, pl.ds(pbase, 16)] = dacc
                nb_v[r, pl.ds(pbase, 16)] = nacc

            rn = r + NBUF

            @pl.when(rn < RPW)
            def _():
                fire(rn, b)

    @pl.loop(0, RPW // 16)
    def _na2_group(rg):
        rows = rg * 16 + _iota16()
        acc = jnp.zeros((16,), jnp.float32)
        for d in range(D):
            col = jnp.full((16,), d, jnp.int32)
            v = plsc.load_gather(hsum_v, [rows, col])
            acc = acc + v * v
        na2_v[pl.ds(rg * 16, 16)] = acc

    pltpu.sync_copy(dot_v, dot_out.at[pl.ds(base, RPW)])
    pltpu.sync_copy(nb_v, nb2_out.at[pl.ds(base, RPW)])
    pltpu.sync_copy(na2_v, na2_out.at[pl.ds(base, RPW)])


def _sc_gather_reduce(table, item_seq, items_to_predict):
    mesh = plsc.VectorSubcoreMesh(core_axis_name="c", subcore_axis_name="s",
                                  num_cores=NC, num_subcores=NS)
    f32 = jnp.float32
    kern = pl.kernel(
        _sc_body,
        out_type=[
            jax.ShapeDtypeStruct((B, NPRED_PAD), f32),
            jax.ShapeDtypeStruct((B, NPRED_PAD), f32),
            jax.ShapeDtypeStruct((B,), f32),
        ],
        mesh=mesh,
        compiler_params=pltpu.CompilerParams(needs_layout_passes=False,
                                             use_tc_tiling_on_sc=False),
        scratch_types=[
            pltpu.VMEM((RPW, HIST), jnp.int32),
            pltpu.VMEM((RPW, NPRED), jnp.int32),
            pltpu.VMEM((RPW, D), f32),
            pltpu.VMEM((16, 16), f32),
            pltpu.VMEM((16, 16), f32),
            pltpu.VMEM((RPW, NPRED_PAD), f32),
            pltpu.VMEM((RPW, NPRED_PAD), f32),
            pltpu.VMEM((RPW,), f32),
        ] + [pltpu.VMEM((HIST, D), f32) for _ in range(NBUF)]
          + [pltpu.VMEM((NPRED_PAD, D), f32) for _ in range(NBUF)]
          + [pltpu.SemaphoreType.DMA for _ in range(NBUF)],
    )
    return kern(table, item_seq, items_to_predict)


ROWS_TC = 256


def _tc_body(seq_ref, dot_ref, nb2_ref, na2_ref, out_ref):
    seq = seq_ref[...]
    denom = jnp.sum((seq != 0).astype(jnp.float32), axis=1, keepdims=True)
    denom = jnp.where(denom == 0.0, 1.0, denom)
    norm_a = jnp.sqrt(na2_ref[...]) / denom + 1e-9
    prod = dot_ref[...] / denom
    out_ref[...] = prod / (norm_a * jnp.sqrt(nb2_ref[...]))


def _tc_normalize(item_seq, dot, nb2, na2):
    grid = (B // ROWS_TC,)
    return pl.pallas_call(
        _tc_body,
        grid=grid,
        in_specs=[
            pl.BlockSpec((ROWS_TC, HIST), lambda i: (i, 0)),
            pl.BlockSpec((ROWS_TC, NPRED_PAD), lambda i: (i, 0)),
            pl.BlockSpec((ROWS_TC, NPRED_PAD), lambda i: (i, 0)),
            pl.BlockSpec((ROWS_TC, 1), lambda i: (i, 0)),
        ],
        out_specs=pl.BlockSpec((ROWS_TC, NPRED_PAD), lambda i: (i, 0)),
        out_shape=jax.ShapeDtypeStruct((B, NPRED_PAD), jnp.float32),
    )(item_seq, dot, nb2, na2.reshape(B, 1))


@jax.jit
def kernel(item_seq, items_to_predict, table):
    seq = item_seq.astype(jnp.int32)
    pidx = items_to_predict.astype(jnp.int32)
    dot, nb2, na2 = _sc_gather_reduce(table, seq, pidx)
    cos = _tc_normalize(seq, dot, nb2, na2)
    return cos[:, :NPRED]


# speedup vs baseline: 2.3930x; 1.8498x over previous
import functools

import jax
import jax.numpy as jnp
from jax import lax
from jax.experimental import pallas as pl
from jax.experimental.pallas import tpu as pltpu
from jax.experimental.pallas import tpu_sc as plsc

B = 4096
IT_BLK = 32768
N_BLK = 31
HALF = N_BLK * IT_BLK // 2
NUM_ROWS = 2 * HALF
HIST = 50
NPRED = 100
NPRED_PAD = 112
D = 64
NC = 2
NS = 16
NW = NC * NS
RPW = B // NW
NBUF = 4
NCH = D // 16


def _iota16():
    return lax.iota(jnp.int32, 16)


def _sc_body(table, seq, pidx, dot_out, nb2_out, na2_out,
             seq_v, pidx_v, hsum_v, tbuf, ubuf, dot_v, nb_v, na2_v,
             h0b, h1b, h2b, h3b, p0b, p1b, p2b, p3b,
             sem0, sem1, sem2, sem3):
    hembs = (h0b, h1b, h2b, h3b)
    pembs = (p0b, p1b, p2b, p3b)
    sems = (sem0, sem1, sem2, sem3)

    wid = lax.axis_index("s") * NC + lax.axis_index("c")
    base = wid * RPW

    pltpu.sync_copy(seq.at[pl.ds(base, RPW)], seq_v)
    pltpu.sync_copy(pidx.at[pl.ds(base, RPW)], pidx_v)

    zeros = jnp.zeros((16,), jnp.float32)

    def fire(r, b):
        pltpu.async_copy(table.at[seq_v.at[r]], hembs[b], sems[b])
        pltpu.async_copy(table.at[pidx_v.at[r]],
                         pembs[b].at[pl.ds(0, NPRED)], sems[b])

    for b in range(NBUF):
        fire(b, b)

    @pl.loop(0, RPW // NBUF)
    def _row_loop(rr):
        for b in range(NBUF):
            r = rr * NBUF + b
            hemb = hembs[b]
            pemb = pembs[b]
            pltpu.make_async_copy(table.at[seq_v.at[r]], hemb,
                                  sems[b]).wait()
            pltpu.make_async_copy(table.at[pidx_v.at[r]],
                                  pemb.at[pl.ds(0, NPRED)], sems[b]).wait()

            def hist_step(i, acc):
                return tuple(acc[c] + hemb[i, pl.ds(c * 16, 16)]
                             for c in range(NCH))

            h = lax.fori_loop(0, HIST, hist_step, (zeros,) * NCH,
                              unroll=5)
            for c in range(NCH):
                hsum_v[r, pl.ds(c * 16, 16)] = h[c]

            rows = _iota16()
            for pg in range(NPRED_PAD // 16):
                pbase = pg * 16
                for j in range(16):
                    p = pbase + j
                    e = [pemb[p, pl.ds(c * 16, 16)] for c in range(NCH)]
                    t = (e[0] * h[0] + e[1] * h[1]) + (e[2] * h[2]
                                                       + e[3] * h[3])
                    u = (e[0] * e[0] + e[1] * e[1]) + (e[2] * e[2]
                                                       + e[3] * e[3])
                    tbuf[j, pl.ds(0, 16)] = t
                    ubuf[j, pl.ds(0, 16)] = u
                dacc = [zeros] * 4
                nacc = [zeros] * 4
                for l in range(16):
                    col = jnp.full((16,), l, jnp.int32)
                    dacc[l % 4] = dacc[l % 4] + plsc.load_gather(
                        tbuf, [rows, col])
                    nacc[l % 4] = nacc[l % 4] + plsc.load_gather(
                        ubuf, [rows, col])
                dot_v[r, pl.ds(pbase, 16)] = ((dacc[0] + dacc[1])
                                              + (dacc[2] + dacc[3]))
                nb_v[r, pl.ds(pbase, 16)] = ((nacc[0] + nacc[1])
                                             + (nacc[2] + nacc[3]))

            rn = r + NBUF

            @pl.when(rn < RPW)
            def _():
                fire(rn, b)

    @pl.loop(0, RPW // 16)
    def _na2_group(rg):
        rows = rg * 16 + _iota16()
        acc = jnp.zeros((16,), jnp.float32)
        for d in range(D):
            col = jnp.full((16,), d, jnp.int32)
            v = plsc.load_gather(hsum_v, [rows, col])
            acc = acc + v * v
        na2_v[pl.ds(rg * 16, 16)] = acc

    pltpu.sync_copy(dot_v, dot_out.at[pl.ds(base, RPW)])
    pltpu.sync_copy(nb_v, nb2_out.at[pl.ds(base, RPW)])
    pltpu.sync_copy(na2_v, na2_out.at[pl.ds(base, RPW)])


def _sc_gather_reduce(table, item_seq, items_to_predict):
    mesh = plsc.VectorSubcoreMesh(core_axis_name="c", subcore_axis_name="s",
                                  num_cores=NC, num_subcores=NS)
    f32 = jnp.float32
    kern = pl.kernel(
        _sc_body,
        out_type=[
            jax.ShapeDtypeStruct((B, NPRED_PAD), f32),
            jax.ShapeDtypeStruct((B, NPRED_PAD), f32),
            jax.ShapeDtypeStruct((B,), f32),
        ],
        mesh=mesh,
        compiler_params=pltpu.CompilerParams(needs_layout_passes=False,
                                             use_tc_tiling_on_sc=False),
        scratch_types=[
            pltpu.VMEM((RPW, HIST), jnp.int32),
            pltpu.VMEM((RPW, NPRED), jnp.int32),
            pltpu.VMEM((RPW, D + 1), f32),
            pltpu.VMEM((16, 17), f32),
            pltpu.VMEM((16, 17), f32),
            pltpu.VMEM((RPW, NPRED_PAD), f32),
            pltpu.VMEM((RPW, NPRED_PAD), f32),
            pltpu.VMEM((RPW,), f32),
        ] + [pltpu.VMEM((HIST, D), f32) for _ in range(NBUF)]
          + [pltpu.VMEM((NPRED_PAD, D), f32) for _ in range(NBUF)]
          + [pltpu.SemaphoreType.DMA for _ in range(NBUF)],
    )
    return kern(table, item_seq, items_to_predict)


def _tr_body(tt_ref, out_ref):
    for m in range(IT_BLK // 1024):
        for p in range(2):
            sub = tt_ref[:, pl.ds((2 * m + p) * 512, 512)]
            out_ref[pl.ds(m * 512, 512), pl.ds(p * D, D)] = sub.T


def _tc_transpose_pack(table):
    tt = table.T
    return pl.pallas_call(
        _tr_body,
        grid=(N_BLK,),
        in_specs=[pl.BlockSpec((D, IT_BLK), lambda i: (0, i))],
        out_specs=pl.BlockSpec((IT_BLK // 2, 128), lambda i: (i, 0)),
        out_shape=jax.ShapeDtypeStruct((HALF, 128), jnp.float32),
    )(tt)


ROWS_TC = 256


def _tc_body(seq_ref, dot_ref, nb2_ref, na2_ref, out_ref):
    seq = seq_ref[...]
    denom = jnp.sum((seq != 0).astype(jnp.float32), axis=1, keepdims=True)
    denom = jnp.where(denom == 0.0, 1.0, denom)
    norm_a = jnp.sqrt(na2_ref[...]) / denom + 1e-9
    prod = dot_ref[...] / denom
    out_ref[...] = prod / (norm_a * jnp.sqrt(nb2_ref[...]))


def _tc_normalize(item_seq, dot, nb2, na2):
    grid = (B // ROWS_TC,)
    return pl.pallas_call(
        _tc_body,
        grid=grid,
        in_specs=[
            pl.BlockSpec((ROWS_TC, HIST), lambda i: (i, 0)),
            pl.BlockSpec((ROWS_TC, NPRED_PAD), lambda i: (i, 0)),
            pl.BlockSpec((ROWS_TC, NPRED_PAD), lambda i: (i, 0)),
            pl.BlockSpec((ROWS_TC, 1), lambda i: (i, 0)),
        ],
        out_specs=pl.BlockSpec((ROWS_TC, NPRED_PAD), lambda i: (i, 0)),
        out_shape=jax.ShapeDtypeStruct((B, NPRED_PAD), jnp.float32),
    )(item_seq, dot, nb2, na2.reshape(B, 1))


@jax.jit
def kernel(item_seq, items_to_predict, table):
    seq = item_seq.astype(jnp.int32)
    pidx = items_to_predict.astype(jnp.int32)
    table_lin = _tc_transpose_pack(table).reshape(NUM_ROWS, D)
    seq_r = (seq // 1024) * 1024 + (seq % 512) * 2 + (seq // 512) % 2
    pidx_r = (pidx // 1024) * 1024 + (pidx % 512) * 2 + (pidx // 512) % 2

    dot, nb2, na2 = _sc_gather_reduce(table_lin, seq_r, pidx_r)
    cos = _tc_normalize(seq, dot, nb2, na2)
    return cos[:, :NPRED]
